# Initial kernel scaffold; baseline (speedup 1.0000x reference)
import jax
import jax.numpy as jnp
from jax.experimental import pallas as pl


def kernel(x, Wr, W1, W2):
    raise NotImplementedError("write your pallas kernel here")


# speedup vs baseline: 2.7225x; 2.7225x over previous
import functools

import jax
import jax.numpy as jnp
from jax.experimental import pallas as pl
from jax.experimental.pallas import tpu as pltpu

_D = 1024
_E = 8
_F = 1024
_K = 2
_N = 2048
_TB = 512
_NB = _N // _TB


def _gelu_tanh(h):
    c = 0.7978845608028654
    return 0.5 * h * (1.0 + jnp.tanh(c * (h + 0.044715 * h * h * h)))


def _moe_kernel(x_ref, wr_ref, w1_ref, w2_ref, out_ref, probs_ref):
    e = pl.program_id(0)
    nb = pl.program_id(1)
    row0 = nb * _TB

    xs = x_ref[pl.ds(row0, _TB), :]

    @pl.when(e == 0)
    def _router():
        logits = jax.lax.dot_general(
            xs, wr_ref[...], (((1,), (1,)), ((), ())),
            preferred_element_type=jnp.float32)
        m = jnp.max(logits, axis=1, keepdims=True)
        ex = jnp.exp(logits - m)
        sm = ex / jnp.sum(ex, axis=1, keepdims=True)
        for j in range(_E):
            sj = sm[:, j:j + 1]
            rank = jnp.sum((sm > sj).astype(jnp.float32), axis=1, keepdims=True)
            if j > 0:
                rank = rank + jnp.sum((sm[:, :j] == sj).astype(jnp.float32),
                                      axis=1, keepdims=True)
            probs_ref[pl.ds(row0, _TB), j:j + 1] = jnp.where(rank < _K, sj, 0.0)

    xb = xs.astype(jnp.bfloat16)
    w1 = w1_ref[0].astype(jnp.bfloat16)
    h = jax.lax.dot_general(xb, w1, (((1,), (1,)), ((), ())),
                            preferred_element_type=jnp.float32)
    h = _gelu_tanh(h)
    w2 = w2_ref[0].astype(jnp.bfloat16)
    y = jax.lax.dot_general(h.astype(jnp.bfloat16), w2, (((1,), (1,)), ((), ())),
                            preferred_element_type=jnp.float32)

    lane = jax.lax.broadcasted_iota(jnp.int32, (_TB, _E), 1)
    pall = probs_ref[pl.ds(row0, _TB), :]
    p = jnp.sum(jnp.where(lane == e, pall, 0.0), axis=1, keepdims=True)

    contrib = p * y

    @pl.when(e == 0)
    def _init():
        out_ref[pl.ds(row0, _TB), :] = contrib

    @pl.when(e > 0)
    def _acc():
        out_ref[pl.ds(row0, _TB), :] = out_ref[pl.ds(row0, _TB), :] + contrib


@functools.partial(jax.jit, static_argnames=("interpret",))
def kernel(x, Wr, W1, W2, interpret=False):
    Bb, Ll, Dd = x.shape
    x2 = x.reshape(-1, Dd)
    out = pl.pallas_call(
        _moe_kernel,
        grid=(_E, _NB),
        in_specs=[
            pl.BlockSpec((_N, _D), lambda e, nb: (0, 0)),
            pl.BlockSpec((_E, _D), lambda e, nb: (0, 0)),
            pl.BlockSpec((1, _F, _D), lambda e, nb: (e, 0, 0)),
            pl.BlockSpec((1, _D, _F), lambda e, nb: (e, 0, 0)),
        ],
        out_specs=pl.BlockSpec((_N, _D), lambda e, nb: (0, 0)),
        out_shape=jax.ShapeDtypeStruct((_N, _D), jnp.float32),
        scratch_shapes=[pltpu.VMEM((_N, _E), jnp.float32)],
        compiler_params=pltpu.CompilerParams(
            dimension_semantics=("arbitrary", "arbitrary")),
        interpret=interpret,
    )(x2, Wr, W1, W2)
    return out.reshape(Bb, Ll, Dd)
